# Initial kernel scaffold; baseline (speedup 1.0000x reference)
import jax
import jax.numpy as jnp
from jax.experimental import pallas as pl


def kernel(x, node2graph, W1, b1, W2, b2):
    raise NotImplementedError("write your pallas kernel here")


# speedup vs baseline: 4.6152x; 4.6152x over previous
import functools

import jax
import jax.numpy as jnp
from jax import lax
from jax.experimental import pallas as pl
from jax.experimental.pallas import tpu as pltpu
from jax.experimental.pallas import tpu_sc as plsc

G = 512
F = 128
V = 100000
OUT_D = 2 * F

NC = 2
NS = 16
NW = NC * NS
GPW = G // NW
TILE = 128
SPAD = 528


def _pool_body(x_hbm, starts_hbm, out_hbm, starts_v, xbuf, pooled_v):
    c = lax.axis_index("c")
    s = lax.axis_index("s")
    wid = c * NS + s
    pltpu.sync_copy(starts_hbm, starts_v)
    g0 = wid * GPW
    va = starts_v[pl.ds(g0, 16)]
    vb = starts_v[pl.ds(g0 + 16, 16)]

    for j in range(GPW):
        row_s = va[j]
        row_e = va[j + 1] if j + 1 < GPW else vb[0]
        n = row_e - row_s

        def tile_body(t, accs):
            rs = row_s + t * TILE
            rs_c = jnp.minimum(rs, V - TILE)
            pltpu.sync_copy(x_hbm.at[pl.ds(rs_c, TILE)], xbuf)
            i_lo = rs - rs_c
            i_hi = jnp.minimum(TILE, row_e - rs_c)

            def row_body(i, accs):
                sums, maxs = accs
                new_s = []
                new_m = []
                for k in range(F // 16):
                    rowk = xbuf[i, pl.ds(k * 16, 16)]
                    new_s.append(sums[k] + rowk)
                    new_m.append(jnp.maximum(maxs[k], rowk))
                return (tuple(new_s), tuple(new_m))

            return lax.fori_loop(i_lo, i_hi, row_body, accs)

        zeros = tuple(jnp.zeros((16,), jnp.float32) for _ in range(F // 16))
        ninf = tuple(jnp.full((16,), -jnp.inf, jnp.float32) for _ in range(F // 16))
        n_tiles = (n + TILE - 1) // TILE
        sums, maxs = lax.fori_loop(0, n_tiles, tile_body, (zeros, ninf))

        cnt_v = jnp.full((16,), jnp.maximum(n, 1).astype(jnp.float32))
        inv_v = 1.0 / cnt_v
        for k in range(F // 16):
            pooled_v[j, pl.ds(k * 16, 16)] = sums[k] * inv_v
            pooled_v[j, pl.ds(F + k * 16, 16)] = maxs[k]

    pltpu.sync_copy(pooled_v, out_hbm.at[pl.ds(g0, GPW)])


@jax.jit
def _sc_pool(x, starts):
    mesh = plsc.VectorSubcoreMesh(core_axis_name="c", subcore_axis_name="s")
    return pl.kernel(
        _pool_body,
        out_type=jax.ShapeDtypeStruct((G, OUT_D), jnp.float32),
        mesh=mesh,
        scratch_types=[
            pltpu.VMEM((SPAD,), jnp.int32),
            pltpu.VMEM((TILE, F), jnp.float32),
            pltpu.VMEM((GPW, OUT_D), jnp.float32),
        ],
        compiler_params=pltpu.CompilerParams(use_tc_tiling_on_sc=False),
    )(x, starts)


def _mlp_body(p_ref, w1_ref, b1_ref, w2_ref, b2_ref, out_ref):
    h = lax.dot_general(p_ref[...], w1_ref[...], (((1,), (1,)), ((), ())),
                        preferred_element_type=jnp.float32)
    h = jnp.maximum(h + b1_ref[...], 0.0)
    o = lax.dot_general(h, w2_ref[...], (((1,), (1,)), ((), ())),
                        preferred_element_type=jnp.float32)
    out_ref[...] = o + b2_ref[...]


@jax.jit
def _mlp(pooled, W1, b1, W2, b2):
    return pl.pallas_call(
        _mlp_body,
        out_shape=jax.ShapeDtypeStruct((G, OUT_D), jnp.float32),
    )(pooled, W1, b1.reshape(1, OUT_D), W2, b2.reshape(1, OUT_D))


def kernel(x, node2graph, W1, b1, W2, b2):
    n2g = node2graph.astype(jnp.int32)
    starts = jnp.searchsorted(n2g, jnp.arange(G + 1, dtype=jnp.int32)).astype(jnp.int32)
    starts = jnp.concatenate([starts, jnp.full((SPAD - G - 1,), V, jnp.int32)])
    pooled = _sc_pool(x, starts)
    out = _mlp(pooled, W1, b1, W2, b2)
    return out


# speedup vs baseline: 5.7031x; 1.2357x over previous
import jax
import jax.numpy as jnp
from jax import lax
from jax.experimental import pallas as pl
from jax.experimental.pallas import tpu as pltpu
from jax.experimental.pallas import tpu_sc as plsc

G = 512
F = 128
V = 100000
OUT_D = 2 * F

NC = 2
NS = 16
NW = NC * NS
GPW = G // NW
TILE = 384
SPAD = 528
UNROLL = 4


def _pool_body(x_hbm, starts_hbm, out_hbm, starts_v, xbuf0, xbuf1, pooled_v,
               sem0, sem1):
    c = lax.axis_index("c")
    s = lax.axis_index("s")
    wid = c * NS + s
    pltpu.sync_copy(starts_hbm, starts_v)
    g0 = wid * GPW
    va = starts_v[pl.ds(g0, 16)]
    ve = starts_v[pl.ds(g0 + 1, 16)]
    S = va[0]
    E = ve[15]
    n_t = (E - S + TILE - 1) // TILE

    def issue(t, buf, sem):
        rs_c = jnp.minimum(S + t * TILE, V - TILE)
        pltpu.make_async_copy(x_hbm.at[pl.ds(rs_c, TILE)], buf, sem).start()

    def wait(buf, sem):
        pltpu.make_async_copy(x_hbm.at[pl.ds(0, TILE)], buf, sem).wait()

    zv = jnp.zeros((16,), jnp.float32)
    nv = jnp.full((16,), -jnp.inf, jnp.float32)
    for j in range(GPW):
        for k in range(F // 16):
            pooled_v[j, pl.ds(k * 16, 16)] = zv
            pooled_v[j, pl.ds(F + k * 16, 16)] = nv


    def process(t, buf):
        glo = S + t * TILE
        rs_c = jnp.minimum(glo, V - TILE)
        ghi = jnp.minimum(glo + TILE, E)
        j_lo = 0
        j_hi = GPW

        def graph_body(j, _):
            b_j = starts_v[pl.ds(g0 + j, 16)][0]
            e_j = starts_v[pl.ds(g0 + j + 1, 16)][0]
            lo = jnp.maximum(b_j, glo) - rs_c
            hi = jnp.minimum(e_j, ghi) - rs_c

            @pl.when(lo < hi)
            def _():
                sums = tuple(pooled_v[j, pl.ds(k * 16, 16)] for k in range(F // 16))
                maxs = tuple(pooled_v[j, pl.ds(F + k * 16, 16)] for k in range(F // 16))

                def rows_body(m, accs, count):
                    sums, maxs = accs
                    new_s = list(sums)
                    new_m = list(maxs)
                    for u in range(count):
                        i = lo + m * count + u if count > 1 else m
                        for k in range(F // 16):
                            rowk = buf[i, pl.ds(k * 16, 16)]
                            new_s[k] = new_s[k] + rowk
                            new_m[k] = jnp.maximum(new_m[k], rowk)
                    return (tuple(new_s), tuple(new_m))

                n_rows = hi - lo
                n_u = n_rows // UNROLL
                accs = lax.fori_loop(
                    0, n_u, lambda m, a: rows_body(m, a, UNROLL), (sums, maxs))
                accs = lax.fori_loop(
                    lo + n_u * UNROLL, hi, lambda i, a: rows_body(i, a, 1), accs)
                sums, maxs = accs
                for k in range(F // 16):
                    pooled_v[j, pl.ds(k * 16, 16)] = sums[k]
                    pooled_v[j, pl.ds(F + k * 16, 16)] = maxs[k]

            return 0

        lax.fori_loop(j_lo, j_hi, graph_body, 0)

    @pl.when(n_t > 0)
    def _():
        issue(0, xbuf0, sem0)

    def pair_body(u, _):
        t0 = 2 * u
        t1 = t0 + 1

        @pl.when(t1 < n_t)
        def _():
            issue(t1, xbuf1, sem1)

        wait(xbuf0, sem0)
        process(t0, xbuf0)

        @pl.when(t0 + 2 < n_t)
        def _():
            issue(t0 + 2, xbuf0, sem0)

        @pl.when(t1 < n_t)
        def _():
            wait(xbuf1, sem1)
            process(t1, xbuf1)

        return 0

    lax.fori_loop(0, (n_t + 1) // 2, pair_body, 0)

    for j in range(GPW):
        b_j = va[j]
        e_j = ve[j]
        cnt_v = jnp.full((16,), jnp.maximum(e_j - b_j, 1).astype(jnp.float32))
        inv_v = 1.0 / cnt_v
        for k in range(F // 16):
            pooled_v[j, pl.ds(k * 16, 16)] = pooled_v[j, pl.ds(k * 16, 16)] * inv_v

    pltpu.sync_copy(pooled_v, out_hbm.at[pl.ds(g0, GPW)])


@jax.jit
def _sc_pool(x, starts):
    mesh = plsc.VectorSubcoreMesh(core_axis_name="c", subcore_axis_name="s")
    return pl.kernel(
        _pool_body,
        out_type=jax.ShapeDtypeStruct((G, OUT_D), jnp.float32),
        mesh=mesh,
        scratch_types=[
            pltpu.VMEM((SPAD,), jnp.int32),
            pltpu.VMEM((TILE, F), jnp.float32),
            pltpu.VMEM((TILE, F), jnp.float32),
            pltpu.VMEM((GPW, OUT_D), jnp.float32),
            pltpu.SemaphoreType.DMA,
            pltpu.SemaphoreType.DMA,
        ],
        compiler_params=pltpu.CompilerParams(use_tc_tiling_on_sc=False),
    )(x, starts)


def _mlp_body(p_ref, w1_ref, b1_ref, w2_ref, b2_ref, out_ref):
    h = lax.dot_general(p_ref[...], w1_ref[...], (((1,), (1,)), ((), ())),
                        preferred_element_type=jnp.float32)
    h = jnp.maximum(h + b1_ref[...], 0.0)
    o = lax.dot_general(h, w2_ref[...], (((1,), (1,)), ((), ())),
                        preferred_element_type=jnp.float32)
    out_ref[...] = o + b2_ref[...]


@jax.jit
def _mlp(pooled, W1, b1, W2, b2):
    return pl.pallas_call(
        _mlp_body,
        out_shape=jax.ShapeDtypeStruct((G, OUT_D), jnp.float32),
    )(pooled, W1, b1.reshape(1, OUT_D), W2, b2.reshape(1, OUT_D))


def kernel(x, node2graph, W1, b1, W2, b2):
    n2g = node2graph.astype(jnp.int32)
    starts = jnp.searchsorted(n2g, jnp.arange(G + 1, dtype=jnp.int32)).astype(jnp.int32)
    starts = jnp.concatenate([starts, jnp.full((SPAD - G - 1,), V, jnp.int32)])
    pooled = _sc_pool(x, starts)
    out = _mlp(pooled, W1, b1, W2, b2)
    return out


# speedup vs baseline: 11.1377x; 1.9529x over previous
import jax
import jax.numpy as jnp
from jax import lax
from jax.experimental import pallas as pl
from jax.experimental.pallas import tpu as pltpu
from jax.experimental.pallas import tpu_sc as plsc

G = 512
F = 128
V = 100000
OUT_D = 2 * F

NC = 2
NS = 16
NW = NC * NS
GPW = G // NW
TILE = 384
SPAD = 544
UNROLL = 4
CH = 3200
CB = CH + 16


def _starts_body(n2g_hbm, part_hbm, buf, part_v):
    c = lax.axis_index("c")
    s = lax.axis_index("s")
    wid = c * NS + s
    base = wid * CH
    base_c = jnp.minimum(base, V - CH)
    pltpu.sync_copy(n2g_hbm.at[pl.ds(base_c, CH)], buf.at[pl.ds(16, CH)])
    buf[pl.ds(0, 16)] = jnp.full((16,), -1, jnp.int32)

    @pl.when(wid > 0)
    def _():
        pltpu.sync_copy(n2g_hbm.at[pl.ds(base_c - 16, 16)], buf.at[pl.ds(0, 16)])

    vfill = jnp.full((16,), V, jnp.int32)
    for r in range(SPAD // 16):
        part_v[pl.ds(r * 16, 16)] = vfill

    ioto = lax.iota(jnp.int32, 16)

    def scan_body(i, _):
        off = 16 + i * 16
        v = buf[pl.ds(off, 16)]
        vp = buf[pl.ds(off - 1, 16)]
        gpos = jnp.full((16,), base_c + i * 16) + ioto
        m = (v != vp) | (gpos == 0)
        m = m & (gpos >= base) & (gpos < V)
        plsc.store_scatter(part_v, [v], gpos, mask=m)
        return 0

    lax.fori_loop(0, CH // 16, scan_body, 0)
    pltpu.sync_copy(part_v, part_hbm.at[wid])


@jax.jit
def _sc_starts(n2g):
    mesh = plsc.VectorSubcoreMesh(core_axis_name="c", subcore_axis_name="s")
    return pl.kernel(
        _starts_body,
        out_type=jax.ShapeDtypeStruct((NW, SPAD), jnp.int32),
        mesh=mesh,
        scratch_types=[
            pltpu.VMEM((CB,), jnp.int32),
            pltpu.VMEM((SPAD,), jnp.int32),
        ],
        compiler_params=pltpu.CompilerParams(use_tc_tiling_on_sc=False, needs_layout_passes=False),
    )(n2g)


def _pool_body(x_hbm, part_hbm, out_hbm, part_all, starts_v, xbuf0, xbuf1,
               pooled_v, sem0, sem1):
    c = lax.axis_index("c")
    s = lax.axis_index("s")
    wid = c * NS + s
    g0 = wid * GPW

    pltpu.sync_copy(part_hbm, part_all)
    for r in range(SPAD // 16):
        m = part_all[0, pl.ds(r * 16, 16)]
        for w in range(1, NW):
            m = jnp.minimum(m, part_all[w, pl.ds(r * 16, 16)])
        starts_v[pl.ds(r * 16, 16)] = m

    carry = jnp.full((16,), V, jnp.int32)
    for r in range(SPAD // 16 - 1, -1, -1):
        v = starts_v[pl.ds(r * 16, 16)]
        bwd = -lax.rev(plsc.cummax(lax.rev(-v, (0,))), (0,))
        out = jnp.minimum(bwd, carry)
        starts_v[pl.ds(r * 16, 16)] = out
        carry = jnp.full((16,), out[0])

    va = starts_v[pl.ds(g0, 16)]
    ve = starts_v[pl.ds(g0 + 1, 16)]
    S = va[0]
    E = ve[15]
    n_t = (E - S + TILE - 1) // TILE

    def issue(t, bufr, sem):
        rs_c = jnp.minimum(S + t * TILE, V - TILE)
        pltpu.make_async_copy(x_hbm.at[pl.ds(rs_c, TILE)], bufr, sem).start()

    def wait(bufr, sem):
        pltpu.make_async_copy(x_hbm.at[pl.ds(0, TILE)], bufr, sem).wait()

    zv = jnp.zeros((16,), jnp.float32)
    nv = jnp.full((16,), -jnp.inf, jnp.float32)
    for j in range(GPW):
        for k in range(F // 16):
            pooled_v[j, pl.ds(k * 16, 16)] = zv
            pooled_v[j, pl.ds(F + k * 16, 16)] = nv

    def process(t, bufr):
        glo = S + t * TILE
        rs_c = jnp.minimum(glo, V - TILE)
        ghi = jnp.minimum(glo + TILE, E)

        def graph_body(j, _):
            b_j = starts_v[pl.ds(g0 + j, 16)][0]
            e_j = starts_v[pl.ds(g0 + j + 1, 16)][0]
            lo = jnp.maximum(b_j, glo) - rs_c
            hi = jnp.minimum(e_j, ghi) - rs_c

            @pl.when(lo < hi)
            def _():
                sums = tuple(pooled_v[j, pl.ds(k * 16, 16)] for k in range(F // 16))
                maxs = tuple(pooled_v[j, pl.ds(F + k * 16, 16)] for k in range(F // 16))

                def rows_body(m, accs, count):
                    sums, maxs = accs
                    new_s = list(sums)
                    new_m = list(maxs)
                    for u in range(count):
                        i = lo + m * count + u if count > 1 else m
                        for k in range(F // 16):
                            rowk = bufr[i, pl.ds(k * 16, 16)]
                            new_s[k] = new_s[k] + rowk
                            new_m[k] = jnp.maximum(new_m[k], rowk)
                    return (tuple(new_s), tuple(new_m))

                n_rows = hi - lo
                n_u = n_rows // UNROLL
                accs = lax.fori_loop(
                    0, n_u, lambda m, a: rows_body(m, a, UNROLL), (sums, maxs))
                accs = lax.fori_loop(
                    lo + n_u * UNROLL, hi, lambda i, a: rows_body(i, a, 1), accs)
                sums, maxs = accs
                for k in range(F // 16):
                    pooled_v[j, pl.ds(k * 16, 16)] = sums[k]
                    pooled_v[j, pl.ds(F + k * 16, 16)] = maxs[k]

            return 0

        lax.fori_loop(0, GPW, graph_body, 0)

    @pl.when(n_t > 0)
    def _():
        issue(0, xbuf0, sem0)

    def pair_body(u, _):
        t0 = 2 * u
        t1 = t0 + 1

        @pl.when(t1 < n_t)
        def _():
            issue(t1, xbuf1, sem1)

        wait(xbuf0, sem0)
        process(t0, xbuf0)

        @pl.when(t0 + 2 < n_t)
        def _():
            issue(t0 + 2, xbuf0, sem0)

        @pl.when(t1 < n_t)
        def _():
            wait(xbuf1, sem1)
            process(t1, xbuf1)

        return 0

    lax.fori_loop(0, (n_t + 1) // 2, pair_body, 0)

    for j in range(GPW):
        b_j = va[j]
        e_j = ve[j]
        cnt_v = jnp.full((16,), jnp.maximum(e_j - b_j, 1).astype(jnp.float32))
        inv_v = 1.0 / cnt_v
        for k in range(F // 16):
            pooled_v[j, pl.ds(k * 16, 16)] = pooled_v[j, pl.ds(k * 16, 16)] * inv_v

    pltpu.sync_copy(pooled_v, out_hbm.at[pl.ds(g0, GPW)])


@jax.jit
def _sc_pool(x, part):
    mesh = plsc.VectorSubcoreMesh(core_axis_name="c", subcore_axis_name="s")
    return pl.kernel(
        _pool_body,
        out_type=jax.ShapeDtypeStruct((G, OUT_D), jnp.float32),
        mesh=mesh,
        scratch_types=[
            pltpu.VMEM((NW, SPAD), jnp.int32),
            pltpu.VMEM((SPAD,), jnp.int32),
            pltpu.VMEM((TILE, F), jnp.float32),
            pltpu.VMEM((TILE, F), jnp.float32),
            pltpu.VMEM((GPW, OUT_D), jnp.float32),
            pltpu.SemaphoreType.DMA,
            pltpu.SemaphoreType.DMA,
        ],
        compiler_params=pltpu.CompilerParams(use_tc_tiling_on_sc=False, needs_layout_passes=False),
    )(x, part)


def _mlp_body(p_ref, w1_ref, b1_ref, w2_ref, b2_ref, out_ref):
    h = lax.dot_general(p_ref[...], w1_ref[...], (((1,), (1,)), ((), ())),
                        preferred_element_type=jnp.float32)
    h = jnp.maximum(h + b1_ref[...], 0.0)
    o = lax.dot_general(h, w2_ref[...], (((1,), (1,)), ((), ())),
                        preferred_element_type=jnp.float32)
    out_ref[...] = o + b2_ref[...]


@jax.jit
def _mlp(pooled, W1, b1, W2, b2):
    return pl.pallas_call(
        _mlp_body,
        out_shape=jax.ShapeDtypeStruct((G, OUT_D), jnp.float32),
    )(pooled, W1, b1.reshape(1, OUT_D), W2, b2.reshape(1, OUT_D))


def kernel(x, node2graph, W1, b1, W2, b2):
    n2g = node2graph.astype(jnp.int32)
    part = _sc_starts(n2g)
    pooled = _sc_pool(x, part)
    out = _mlp(pooled, W1, b1, W2, b2)
    return out


# speedup vs baseline: 12.6676x; 1.1374x over previous
import jax
import jax.numpy as jnp
from jax import lax
from jax.experimental import pallas as pl
from jax.experimental.pallas import tpu as pltpu
from jax.experimental.pallas import tpu_sc as plsc

G = 512
F = 128
V = 100000
OUT_D = 2 * F

NC = 2
NS = 16
NW = NC * NS
GPW = G // NW
TILE = 384
SPAD = 544
UNROLL = 4
CH = 6256
CB = CH + 16


def _pool_body(x_hbm, n2g_hbm, out_hbm, buf, part_v, part_sh, part_all,
               starts_v, xbuf0, xbuf1, pooled_v, sem0, sem1):
    c = lax.axis_index("c")
    s = lax.axis_index("s")
    wid = c * NS + s
    g0 = wid * GPW

    base = s * CH
    base_c = jnp.minimum(base, V - CH)
    pltpu.sync_copy(n2g_hbm.at[pl.ds(base_c, CH)], buf.at[pl.ds(16, CH)])
    buf[pl.ds(0, 16)] = jnp.full((16,), -1, jnp.int32)

    @pl.when(s > 0)
    def _():
        pltpu.sync_copy(n2g_hbm.at[pl.ds(base_c - 16, 16)], buf.at[pl.ds(0, 16)])

    vfill = jnp.full((16,), V, jnp.int32)
    for r in range(SPAD // 16):
        part_v[pl.ds(r * 16, 16)] = vfill

    ioto = lax.iota(jnp.int32, 16)

    def scan_body(i, _):
        off = 16 + i * 16
        v = buf[pl.ds(off, 16)]
        vp = buf[pl.ds(off - 1, 16)]
        gpos = jnp.full((16,), base_c + i * 16) + ioto
        m = (v != vp) | (gpos == 0)
        m = m & (gpos >= base) & (gpos < V)
        plsc.store_scatter(part_v, [v], gpos, mask=m)
        return 0

    lax.fori_loop(0, CH // 16, scan_body, 0)

    pltpu.sync_copy(part_v, part_sh.at[s])
    plsc.subcore_barrier()
    pltpu.sync_copy(part_sh, part_all)
    for r in range(SPAD // 16):
        m = part_all[0, pl.ds(r * 16, 16)]
        for w in range(1, NS):
            m = jnp.minimum(m, part_all[w, pl.ds(r * 16, 16)])
        starts_v[pl.ds(r * 16, 16)] = m

    carry = jnp.full((16,), V, jnp.int32)
    for r in range(SPAD // 16 - 1, -1, -1):
        v = starts_v[pl.ds(r * 16, 16)]
        bwd = -lax.rev(plsc.cummax(lax.rev(-v, (0,))), (0,))
        out = jnp.minimum(bwd, carry)
        starts_v[pl.ds(r * 16, 16)] = out
        carry = jnp.full((16,), out[0])

    va = starts_v[pl.ds(g0, 16)]
    ve = starts_v[pl.ds(g0 + 1, 16)]
    S = va[0]
    E = ve[15]
    n_t = (E - S + TILE - 1) // TILE

    def issue(t, bufr, sem):
        rs_c = jnp.minimum(S + t * TILE, V - TILE)
        pltpu.make_async_copy(x_hbm.at[pl.ds(rs_c, TILE)], bufr, sem).start()

    def wait(bufr, sem):
        pltpu.make_async_copy(x_hbm.at[pl.ds(0, TILE)], bufr, sem).wait()

    zv = jnp.zeros((16,), jnp.float32)
    nv = jnp.full((16,), -jnp.inf, jnp.float32)
    for j in range(GPW):
        for k in range(F // 16):
            pooled_v[j, pl.ds(k * 16, 16)] = zv
            pooled_v[j, pl.ds(F + k * 16, 16)] = nv

    def process(t, bufr):
        glo = S + t * TILE
        rs_c = jnp.minimum(glo, V - TILE)
        ghi = jnp.minimum(glo + TILE, E)

        def graph_body(j, _):
            b_j = starts_v[pl.ds(g0 + j, 16)][0]
            e_j = starts_v[pl.ds(g0 + j + 1, 16)][0]
            lo = jnp.maximum(b_j, glo) - rs_c
            hi = jnp.minimum(e_j, ghi) - rs_c

            @pl.when(lo < hi)
            def _():
                sums = tuple(pooled_v[j, pl.ds(k * 16, 16)] for k in range(F // 16))
                maxs = tuple(pooled_v[j, pl.ds(F + k * 16, 16)] for k in range(F // 16))

                def rows_body(m, accs, count):
                    sums, maxs = accs
                    new_s = list(sums)
                    new_m = list(maxs)
                    for u in range(count):
                        i = lo + m * count + u if count > 1 else m
                        for k in range(F // 16):
                            rowk = bufr[i, pl.ds(k * 16, 16)]
                            new_s[k] = new_s[k] + rowk
                            new_m[k] = jnp.maximum(new_m[k], rowk)
                    return (tuple(new_s), tuple(new_m))

                n_rows = hi - lo
                n_u = n_rows // UNROLL
                accs = lax.fori_loop(
                    0, n_u, lambda m, a: rows_body(m, a, UNROLL), (sums, maxs))
                accs = lax.fori_loop(
                    lo + n_u * UNROLL, hi, lambda i, a: rows_body(i, a, 1), accs)
                sums, maxs = accs
                for k in range(F // 16):
                    pooled_v[j, pl.ds(k * 16, 16)] = sums[k]
                    pooled_v[j, pl.ds(F + k * 16, 16)] = maxs[k]

            return 0

        lax.fori_loop(0, GPW, graph_body, 0)

    @pl.when(n_t > 0)
    def _():
        issue(0, xbuf0, sem0)

    def pair_body(u, _):
        t0 = 2 * u
        t1 = t0 + 1

        @pl.when(t1 < n_t)
        def _():
            issue(t1, xbuf1, sem1)

        wait(xbuf0, sem0)
        process(t0, xbuf0)

        @pl.when(t0 + 2 < n_t)
        def _():
            issue(t0 + 2, xbuf0, sem0)

        @pl.when(t1 < n_t)
        def _():
            wait(xbuf1, sem1)
            process(t1, xbuf1)

        return 0

    lax.fori_loop(0, (n_t + 1) // 2, pair_body, 0)

    for j in range(GPW):
        b_j = va[j]
        e_j = ve[j]
        cnt_v = jnp.full((16,), jnp.maximum(e_j - b_j, 1).astype(jnp.float32))
        inv_v = 1.0 / cnt_v
        for k in range(F // 16):
            pooled_v[j, pl.ds(k * 16, 16)] = pooled_v[j, pl.ds(k * 16, 16)] * inv_v

    pltpu.sync_copy(pooled_v, out_hbm.at[pl.ds(g0, GPW)])


@jax.jit
def _sc_pool(x, n2g):
    mesh = plsc.VectorSubcoreMesh(core_axis_name="c", subcore_axis_name="s")
    return pl.kernel(
        _pool_body,
        out_type=jax.ShapeDtypeStruct((G, OUT_D), jnp.float32),
        mesh=mesh,
        scratch_types=[
            pltpu.VMEM((CB,), jnp.int32),
            pltpu.VMEM((SPAD,), jnp.int32),
            pltpu.VMEM_SHARED((NS, SPAD), jnp.int32),
            pltpu.VMEM((NS, SPAD), jnp.int32),
            pltpu.VMEM((SPAD,), jnp.int32),
            pltpu.VMEM((TILE, F), jnp.float32),
            pltpu.VMEM((TILE, F), jnp.float32),
            pltpu.VMEM((GPW, OUT_D), jnp.float32),
            pltpu.SemaphoreType.DMA,
            pltpu.SemaphoreType.DMA,
        ],
        compiler_params=pltpu.CompilerParams(use_tc_tiling_on_sc=False, needs_layout_passes=False),
    )(x, n2g)


def _mlp_body(p_ref, w1_ref, b1_ref, w2_ref, b2_ref, out_ref):
    h = lax.dot_general(p_ref[...], w1_ref[...], (((1,), (1,)), ((), ())),
                        preferred_element_type=jnp.float32)
    h = jnp.maximum(h + b1_ref[...], 0.0)
    o = lax.dot_general(h, w2_ref[...], (((1,), (1,)), ((), ())),
                        preferred_element_type=jnp.float32)
    out_ref[...] = o + b2_ref[...]


@jax.jit
def _mlp(pooled, W1, b1, W2, b2):
    return pl.pallas_call(
        _mlp_body,
        out_shape=jax.ShapeDtypeStruct((G, OUT_D), jnp.float32),
    )(pooled, W1, b1.reshape(1, OUT_D), W2, b2.reshape(1, OUT_D))


def kernel(x, node2graph, W1, b1, W2, b2):
    n2g = node2graph.astype(jnp.int32)
    pooled = _sc_pool(x, n2g)
    out = _mlp(pooled, W1, b1, W2, b2)
    return out


# speedup vs baseline: 12.9490x; 1.0222x over previous
import jax
import jax.numpy as jnp
from jax import lax
from jax.experimental import pallas as pl
from jax.experimental.pallas import tpu as pltpu
from jax.experimental.pallas import tpu_sc as plsc

G = 512
F = 128
V = 100000
OUT_D = 2 * F

NC = 2
NS = 16
NW = NC * NS
GPW = G // NW
TILE = 416
SPAD = 544
UNROLL = 4
CH = 6256
CB = CH + 16


def _pool_body(x_hbm, n2g_hbm, out_hbm, buf, part_v, part_sh, part_all,
               starts_v, xbuf0, xbuf1, pooled_v, sem0, sem1):
    c = lax.axis_index("c")
    s = lax.axis_index("s")
    wid = c * NS + s
    g0 = wid * GPW

    base = s * CH
    base_c = jnp.minimum(base, V - CH)
    pltpu.sync_copy(n2g_hbm.at[pl.ds(base_c, CH)], buf.at[pl.ds(16, CH)])
    buf[pl.ds(0, 16)] = jnp.full((16,), -1, jnp.int32)

    @pl.when(s > 0)
    def _():
        pltpu.sync_copy(n2g_hbm.at[pl.ds(base_c - 16, 16)], buf.at[pl.ds(0, 16)])

    vfill = jnp.full((16,), V, jnp.int32)
    for r in range(SPAD // 16):
        part_v[pl.ds(r * 16, 16)] = vfill

    ioto = lax.iota(jnp.int32, 16)

    def scan_body(i, _):
        off = 16 + i * 16
        v = buf[pl.ds(off, 16)]
        vp = buf[pl.ds(off - 1, 16)]
        gpos = jnp.full((16,), base_c + i * 16) + ioto
        m = (v != vp) | (gpos == 0)
        m = m & (gpos >= base) & (gpos < V)
        plsc.store_scatter(part_v, [v], gpos, mask=m)
        return 0

    lax.fori_loop(0, CH // 16, scan_body, 0)

    pltpu.sync_copy(part_v, part_sh.at[s])
    plsc.subcore_barrier()
    pltpu.sync_copy(part_sh, part_all)
    for r in range(SPAD // 16):
        m = part_all[0, pl.ds(r * 16, 16)]
        for w in range(1, NS):
            m = jnp.minimum(m, part_all[w, pl.ds(r * 16, 16)])
        starts_v[pl.ds(r * 16, 16)] = m

    carry = jnp.full((16,), V, jnp.int32)
    for r in range(SPAD // 16 - 1, -1, -1):
        v = starts_v[pl.ds(r * 16, 16)]
        bwd = -lax.rev(plsc.cummax(lax.rev(-v, (0,))), (0,))
        out = jnp.minimum(bwd, carry)
        starts_v[pl.ds(r * 16, 16)] = out
        carry = jnp.full((16,), out[0])

    va = starts_v[pl.ds(g0, 16)]
    ve = starts_v[pl.ds(g0 + 1, 16)]
    S = va[0]
    E = ve[15]
    n_t = (E - S + TILE - 1) // TILE

    def issue(t, bufr, sem):
        rs_c = jnp.minimum(S + t * TILE, V - TILE)
        pltpu.make_async_copy(x_hbm.at[pl.ds(rs_c, TILE)], bufr, sem).start()

    def wait(bufr, sem):
        pltpu.make_async_copy(x_hbm.at[pl.ds(0, TILE)], bufr, sem).wait()

    zv = jnp.zeros((16,), jnp.float32)
    nv = jnp.full((16,), -jnp.inf, jnp.float32)
    for j in range(GPW):
        for k in range(F // 16):
            pooled_v[j, pl.ds(k * 16, 16)] = zv
            pooled_v[j, pl.ds(F + k * 16, 16)] = nv

    def process(t, bufr):
        glo = S + t * TILE
        rs_c = jnp.minimum(glo, V - TILE)
        ghi = jnp.minimum(glo + TILE, E)
        j_lo = plsc.all_reduce_population_count(ve <= jnp.full((16,), glo))[0]
        j_hi = plsc.all_reduce_population_count(va < jnp.full((16,), ghi))[0]

        def graph_body(j, _):
            b_j = starts_v[pl.ds(g0 + j, 16)][0]
            e_j = starts_v[pl.ds(g0 + j + 1, 16)][0]
            lo = jnp.maximum(b_j, glo) - rs_c
            hi = jnp.minimum(e_j, ghi) - rs_c

            @pl.when(lo < hi)
            def _():
                sums = tuple(pooled_v[j, pl.ds(k * 16, 16)] for k in range(F // 16))
                maxs = tuple(pooled_v[j, pl.ds(F + k * 16, 16)] for k in range(F // 16))

                def rows_body(m, accs, count):
                    sums, maxs = accs
                    new_s = list(sums)
                    new_m = list(maxs)
                    for u in range(count):
                        i = lo + m * count + u if count > 1 else m
                        for k in range(F // 16):
                            rowk = bufr[i, pl.ds(k * 16, 16)]
                            new_s[k] = new_s[k] + rowk
                            new_m[k] = jnp.maximum(new_m[k], rowk)
                    return (tuple(new_s), tuple(new_m))

                n_rows = hi - lo
                n_u = n_rows // UNROLL
                accs = lax.fori_loop(
                    0, n_u, lambda m, a: rows_body(m, a, UNROLL), (sums, maxs))
                accs = lax.fori_loop(
                    lo + n_u * UNROLL, hi, lambda i, a: rows_body(i, a, 1), accs)
                sums, maxs = accs
                for k in range(F // 16):
                    pooled_v[j, pl.ds(k * 16, 16)] = sums[k]
                    pooled_v[j, pl.ds(F + k * 16, 16)] = maxs[k]

            return 0

        lax.fori_loop(j_lo, j_hi, graph_body, 0)

    @pl.when(n_t > 0)
    def _():
        issue(0, xbuf0, sem0)

    def pair_body(u, _):
        t0 = 2 * u
        t1 = t0 + 1

        @pl.when(t1 < n_t)
        def _():
            issue(t1, xbuf1, sem1)

        wait(xbuf0, sem0)
        process(t0, xbuf0)

        @pl.when(t0 + 2 < n_t)
        def _():
            issue(t0 + 2, xbuf0, sem0)

        @pl.when(t1 < n_t)
        def _():
            wait(xbuf1, sem1)
            process(t1, xbuf1)

        return 0

    lax.fori_loop(0, (n_t + 1) // 2, pair_body, 0)

    for j in range(GPW):
        b_j = va[j]
        e_j = ve[j]
        cnt_v = jnp.full((16,), jnp.maximum(e_j - b_j, 1).astype(jnp.float32))
        inv_v = 1.0 / cnt_v
        for k in range(F // 16):
            pooled_v[j, pl.ds(k * 16, 16)] = pooled_v[j, pl.ds(k * 16, 16)] * inv_v

    pltpu.sync_copy(pooled_v, out_hbm.at[pl.ds(g0, GPW)])


@jax.jit
def _sc_pool(x, n2g):
    mesh = plsc.VectorSubcoreMesh(core_axis_name="c", subcore_axis_name="s")
    return pl.kernel(
        _pool_body,
        out_type=jax.ShapeDtypeStruct((G, OUT_D), jnp.float32),
        mesh=mesh,
        scratch_types=[
            pltpu.VMEM((CB,), jnp.int32),
            pltpu.VMEM((SPAD,), jnp.int32),
            pltpu.VMEM_SHARED((NS, SPAD), jnp.int32),
            pltpu.VMEM((NS, SPAD), jnp.int32),
            pltpu.VMEM((SPAD,), jnp.int32),
            pltpu.VMEM((TILE, F), jnp.float32),
            pltpu.VMEM((TILE, F), jnp.float32),
            pltpu.VMEM((GPW, OUT_D), jnp.float32),
            pltpu.SemaphoreType.DMA,
            pltpu.SemaphoreType.DMA,
        ],
        compiler_params=pltpu.CompilerParams(use_tc_tiling_on_sc=False, needs_layout_passes=False),
    )(x, n2g)


def _mlp_body(p_ref, w1_ref, b1_ref, w2_ref, b2_ref, out_ref):
    h = lax.dot_general(p_ref[...], w1_ref[...], (((1,), (1,)), ((), ())),
                        preferred_element_type=jnp.float32)
    h = jnp.maximum(h + b1_ref[...], 0.0)
    o = lax.dot_general(h, w2_ref[...], (((1,), (1,)), ((), ())),
                        preferred_element_type=jnp.float32)
    out_ref[...] = o + b2_ref[...]


@jax.jit
def _mlp(pooled, W1, b1, W2, b2):
    return pl.pallas_call(
        _mlp_body,
        out_shape=jax.ShapeDtypeStruct((G, OUT_D), jnp.float32),
    )(pooled, W1, b1.reshape(1, OUT_D), W2, b2.reshape(1, OUT_D))


def kernel(x, node2graph, W1, b1, W2, b2):
    n2g = node2graph.astype(jnp.int32)
    pooled = _sc_pool(x, n2g)
    out = _mlp(pooled, W1, b1, W2, b2)
    return out
